# baseline (speedup 1.0000x reference)
import functools

import jax
import jax.numpy as jnp
from jax import lax
from jax.experimental import pallas as pl
from jax.experimental.pallas import tpu as pltpu
from jax.experimental.pallas import tpu_sc as plsc

NC = 2
NS = 16
NW = NC * NS
CH = 128


def _emb_gather(weight, idx2):
    n_blocks, ch = idx2.shape
    _, D = weight.shape
    blocks_per_w = n_blocks // NW
    mesh = plsc.VectorSubcoreMesh(core_axis_name="c", subcore_axis_name="s")

    @functools.partial(
        pl.kernel,
        out_type=jax.ShapeDtypeStruct((n_blocks * ch, D), jnp.float32),
        mesh=mesh,
        compiler_params=pltpu.CompilerParams(use_tc_tiling_on_sc=False),
        scratch_types=[
            pltpu.VMEM((blocks_per_w, ch), jnp.int32),
            pltpu.VMEM((ch, D), jnp.float32),
            pltpu.SemaphoreType.DMA,
        ],
    )
    def k(w_hbm, idx_hbm, out_hbm, idx_v, rows_v, sem):
        wid = lax.axis_index("s") * NC + lax.axis_index("c")
        blk0 = wid * blocks_per_w
        pltpu.sync_copy(idx_hbm.at[pl.ds(blk0, blocks_per_w)], idx_v)

        def body(c, carry):
            pltpu.async_copy(w_hbm.at[idx_v.at[c]], rows_v, sem).wait()
            pltpu.sync_copy(rows_v, out_hbm.at[pl.ds((blk0 + c) * ch, ch)])
            return carry

        lax.fori_loop(0, blocks_per_w, body, 0)

    return k(weight, idx2)


def kernel(x, weight):
    B, S = x.shape
    D = weight.shape[1]
    N = B * S
    idx2 = x.reshape(N // CH, CH).astype(jnp.int32)
    out = _emb_gather(weight, idx2)
    return out.reshape(B, S, D)


# speedup vs baseline: 1.1169x; 1.1169x over previous
import functools

import jax
import jax.numpy as jnp
from jax import lax
from jax.experimental import pallas as pl
from jax.experimental.pallas import tpu as pltpu
from jax.experimental.pallas import tpu_sc as plsc

NC = 2
NS = 16
NW = NC * NS
CH = 128
K = 4


def _emb_gather(weight, idx2):
    n_blocks, ch = idx2.shape
    _, D = weight.shape
    blocks_per_w = n_blocks // NW
    ngroups = blocks_per_w // K
    mesh = plsc.VectorSubcoreMesh(core_axis_name="c", subcore_axis_name="s")

    @functools.partial(
        pl.kernel,
        out_type=jax.ShapeDtypeStruct((n_blocks * ch, D), jnp.float32),
        mesh=mesh,
        compiler_params=pltpu.CompilerParams(use_tc_tiling_on_sc=False),
        scratch_types=[
            pltpu.VMEM((blocks_per_w, ch), jnp.int32),
            pltpu.VMEM((K, ch, D), jnp.float32),
            pltpu.VMEM((K, ch, D), jnp.float32),
            pltpu.SemaphoreType.DMA,
            pltpu.SemaphoreType.DMA,
            pltpu.SemaphoreType.DMA,
            pltpu.SemaphoreType.DMA,
        ],
    )
    def k(w_hbm, idx_hbm, out_hbm, idx_v, rows_a, rows_b,
          gsem_a, gsem_b, wsem_a, wsem_b):
        wid = lax.axis_index("s") * NC + lax.axis_index("c")
        blk0 = wid * blocks_per_w
        pltpu.sync_copy(idx_hbm.at[pl.ds(blk0, blocks_per_w)], idx_v)

        banks = ((rows_a, gsem_a, wsem_a), (rows_b, gsem_b, wsem_b))

        def gather_start(bank, local_blk, b):
            rows, gsem, _ = banks[bank]
            pltpu.async_copy(w_hbm.at[idx_v.at[local_blk]], rows.at[b], gsem)

        def gather_wait(bank, local_blk, b):
            rows, gsem, _ = banks[bank]
            pltpu.make_async_copy(
                w_hbm.at[idx_v.at[local_blk]], rows.at[b], gsem).wait()

        def write_start(bank, local_blk, b):
            rows, _, wsem = banks[bank]
            dst = out_hbm.at[pl.ds((blk0 + local_blk) * ch, ch)]
            pltpu.async_copy(rows.at[b], dst, wsem)

        def write_wait(bank, local_blk, b):
            rows, _, wsem = banks[bank]
            dst = out_hbm.at[pl.ds((blk0 + local_blk) * ch, ch)]
            pltpu.make_async_copy(rows.at[b], dst, wsem).wait()

        for b in range(K):
            gather_start(0, b, b)

        @pl.loop(0, ngroups, step=2)
        def _(g):
            for p in range(2):
                gp = g + p

                @pl.when(gp + 1 < ngroups)
                def _():
                    @pl.when(gp >= 1)
                    def _():
                        for b in range(K):
                            write_wait(1 - p, (gp - 1) * K + b, b)
                    for b in range(K):
                        gather_start(1 - p, (gp + 1) * K + b, b)

                for b in range(K):
                    gather_wait(p, gp * K + b, b)
                for b in range(K):
                    write_start(p, gp * K + b, b)

        for b in range(K):
            write_wait((ngroups - 1) % 2, (ngroups - 1) * K + b, b)

    return k(weight, idx2)


def kernel(x, weight):
    B, S = x.shape
    D = weight.shape[1]
    N = B * S
    idx2 = x.reshape(N // CH, CH).astype(jnp.int32)
    out = _emb_gather(weight, idx2)
    return out.reshape(B, S, D)
